# baseline (speedup 1.0000x reference)
import jax
import jax.numpy as jnp
from jax.experimental import pallas as pl

N, J, D = 10000, 32, 128
TCB = 480
INV = 1.0 / J


def _mean_body(x_ref, o_ref):
    o_ref[...] = jnp.sum(x_ref[...], axis=1) * INV


_mean = pl.pallas_call(
    _mean_body,
    grid=(pl.cdiv(N, TCB),),
    in_specs=[pl.BlockSpec((TCB, J, D), lambda i: (i, 0, 0))],
    out_specs=pl.BlockSpec((TCB, D), lambda i: (i, 0)),
    out_shape=jax.ShapeDtypeStruct((N, D), jnp.float32),
)


def kernel(neighbours_features):
    return _mean(neighbours_features)


# speedup vs baseline: 1.0042x; 1.0042x over previous
import jax
import jax.numpy as jnp
from jax.experimental import pallas as pl

N, J, D = 10000, 32, 128
TCB = 400
INV = 1.0 / J


def _mean_body(x_ref, o_ref):
    o_ref[...] = jnp.sum(x_ref[...], axis=1) * INV


_mean = pl.pallas_call(
    _mean_body,
    grid=(pl.cdiv(N, TCB),),
    in_specs=[pl.BlockSpec((TCB, J, D), lambda i: (i, 0, 0))],
    out_specs=pl.BlockSpec((TCB, D), lambda i: (i, 0)),
    out_shape=jax.ShapeDtypeStruct((N, D), jnp.float32),
)


def kernel(neighbours_features):
    return _mean(neighbours_features)
